# Initial kernel scaffold; baseline (speedup 1.0000x reference)
import functools

import jax
import jax.numpy as jnp
from jax import lax
from jax.experimental import pallas as pl
from jax.experimental.pallas import tpu as pltpu
from jax.experimental.pallas import tpu_sc as plsc

_B = 16384
_F = 26
_V = 100000
_D = 16
_NF = _F + 1
_NPAIR = _NF * (_NF - 1) // 2

_NW = 32
_ROWS = _F * _B
_RPW = _ROWS // _NW
_TR = 128
_NT = _RPW // _TR
_K = 8
_NSUP = _NT // _K

_BBLK = 512


def _gather_body(table_hbm, idx_hbm, out_hbm, idx_v, rows_v, gsem):
    wid = lax.axis_index("s") * 2 + lax.axis_index("c")
    wbase = wid * _RPW
    pltpu.sync_copy(idx_hbm.at[wid], idx_v)

    def super_body(s, carry):
        base = s * _K
        copies = []
        for b in range(_K):
            copies.append(
                pltpu.async_copy(table_hbm.at[idx_v.at[base + b]],
                                 rows_v.at[b], gsem))
        for cp in copies:
            cp.wait()
        for b in range(_K):
            off = pl.multiple_of(wbase + (base + b) * _TR, 8)
            pltpu.sync_copy(rows_v.at[b], out_hbm.at[pl.ds(off, _TR)])
        return carry

    lax.fori_loop(0, _NSUP, super_body, 0)


def _sc_gather(table_flat, idx3):
    mesh = plsc.VectorSubcoreMesh(core_axis_name="c", subcore_axis_name="s")
    k = functools.partial(
        pl.kernel,
        mesh=mesh,
        out_type=jax.ShapeDtypeStruct((_ROWS, _D), jnp.float32),
        scratch_types=[
            pltpu.VMEM((_NT, _TR), jnp.int32),
            pltpu.VMEM((_K, _TR, _D), jnp.float32),
            pltpu.SemaphoreType.DMA,
        ],
    )(_gather_body)
    return k(table_flat, idx3)


def _dense_body(emb_ref, price_ref, wp_ref, bp_ref, w1_ref, b1_ref, w2_ref,
                b2_ref, mi_ref, wtd_ref, bt1_ref, wt2_ref, bt2_ref,
                out_ref, ft_ref, g_ref):
    de = wp_ref[...] * price_ref[...] + bp_ref[...]
    for f in range(_F):
        ft_ref[f] = emb_ref[f].T
    ft_ref[_F] = de
    ft = ft_ref[...]
    off = 0
    for i in range(_NF - 1):
        cnt = _NF - 1 - i
        gi = jnp.sum(ft[i + 1:] * ft[i:i + 1], axis=1)
        g_ref[off:off + cnt] = gi
        off += cnt
    h = jnp.maximum(
        jnp.dot(w1_ref[...], de, preferred_element_type=jnp.float32)
        + b1_ref[...], 0.0)
    di = jnp.maximum(
        jnp.dot(w2_ref[...], h, preferred_element_type=jnp.float32)
        + b2_ref[...], 0.0)
    tp = (jnp.dot(mi_ref[...], g_ref[...], preferred_element_type=jnp.float32)
          + jnp.dot(wtd_ref[...], di, preferred_element_type=jnp.float32)
          + bt1_ref[...])
    t = jnp.maximum(tp, 0.0)
    out_ref[...] = (jnp.dot(wt2_ref[...], t, preferred_element_type=jnp.float32)
                    + bt2_ref[...])


def _tc_dense(emb, price2, wp, bp2, w1, b12, w2, b22, m_int, wt1d, bt12,
              wt2, bt22):
    nblk = _B // _BBLK
    full = lambda a: pl.BlockSpec(a.shape, lambda i: tuple(0 for _ in a.shape))
    out2 = pl.pallas_call(
        _dense_body,
        grid=(nblk,),
        in_specs=[
            pl.BlockSpec((_F, _BBLK, _D), lambda i: (0, i, 0)),
            pl.BlockSpec((1, _BBLK), lambda i: (0, i)),
            full(wp), full(bp2), full(w1), full(b12),
            full(w2), full(b22), full(m_int),
            full(wt1d), full(bt12), full(wt2), full(bt22),
        ],
        out_specs=pl.BlockSpec((1, _BBLK), lambda i: (0, i)),
        out_shape=jax.ShapeDtypeStruct((1, _B), jnp.float32),
        scratch_shapes=[
            pltpu.VMEM((_NF, _D, _BBLK), jnp.float32),
            pltpu.VMEM((_NPAIR, _BBLK), jnp.float32),
        ],
    )(emb, price2, wp, bp2, w1, b12, w2, b22, m_int, wt1d, bt12, wt2, bt22)
    return out2


def kernel(x_cat, price, tables, W_price, b_price, W1, b1, W2, b2,
           Wt1, bt1, Wt2, bt2):
    offs = (jnp.arange(_F, dtype=jnp.int32) * _V)[:, None]
    idx3 = (x_cat.astype(jnp.int32) + offs).reshape(_NW, _NT, _TR)
    table_flat = tables.reshape(_F * _V, _D)
    emb_flat = _sc_gather(table_flat, idx3)
    emb = emb_flat.reshape(_F, _B, _D)
    out2 = _tc_dense(
        emb,
        price.reshape(1, _B),
        W_price,
        b_price.reshape(_D, 1),
        W1, b1.reshape(-1, 1),
        W2, b2.reshape(-1, 1),
        Wt1[:, :_NPAIR],
        Wt1[:, _NPAIR:],
        bt1.reshape(-1, 1),
        Wt2,
        bt2.reshape(1, 1),
    )
    return out2.reshape(_B)


# speedup vs baseline: 7.0240x; 7.0240x over previous
import functools

import jax
import jax.numpy as jnp
from jax import lax
from jax.experimental import pallas as pl
from jax.experimental.pallas import tpu as pltpu
from jax.experimental.pallas import tpu_sc as plsc

_B = 16384
_F = 26
_V = 100000
_D = 16
_NF = _F + 1
_NPAIR = _NF * (_NF - 1) // 2

_NW = 32
_ROWS = _F * _B
_RPW = _ROWS // _NW
_TR = 128
_NT = _RPW // _TR
_K = 8
_NSUP = _NT // _K

_BBLK = 512


def _gather_body(table_hbm, idx_hbm, out_hbm, idx_v, rows_v, gsem):
    wid = lax.axis_index("s") * 2 + lax.axis_index("c")
    wbase = wid * _RPW
    pltpu.sync_copy(idx_hbm.at[wid], idx_v)

    def super_body(s, carry):
        base = s * _K
        copies = []
        for b in range(_K):
            copies.append(
                pltpu.async_copy(table_hbm.at[idx_v.at[base + b]],
                                 rows_v.at[b], gsem))
        for cp in copies:
            cp.wait()
        for b in range(_K):
            off = pl.multiple_of(wbase + (base + b) * _TR, 8)
            pltpu.sync_copy(rows_v.at[b], out_hbm.at[pl.ds(off, _TR)])
        return carry

    lax.fori_loop(0, _NSUP, super_body, 0)


def _sc_gather(table_flat, idx3):
    mesh = plsc.VectorSubcoreMesh(core_axis_name="c", subcore_axis_name="s")
    k = functools.partial(
        pl.kernel,
        mesh=mesh,
        out_type=jax.ShapeDtypeStruct((_ROWS, _D), jnp.float32),
        scratch_types=[
            pltpu.VMEM((_NT, _TR), jnp.int32),
            pltpu.VMEM((_K, _TR, _D), jnp.float32),
            pltpu.SemaphoreType.DMA,
        ],
        compiler_params=pltpu.CompilerParams(use_tc_tiling_on_sc=False),
    )(_gather_body)
    return k(table_flat, idx3)


def _dense_body(emb_ref, price_ref, wp_ref, bp_ref, w1_ref, b1_ref, w2_ref,
                b2_ref, mi_ref, wtd_ref, bt1_ref, wt2_ref, bt2_ref,
                out_ref, ft_ref, g_ref):
    de = wp_ref[...] * price_ref[...] + bp_ref[...]
    for f in range(_F):
        ft_ref[f] = emb_ref[f].T
    ft_ref[_F] = de
    ft = ft_ref[...]
    off = 0
    for i in range(_NF - 1):
        cnt = _NF - 1 - i
        gi = jnp.sum(ft[i + 1:] * ft[i:i + 1], axis=1)
        g_ref[off:off + cnt] = gi
        off += cnt
    h = jnp.maximum(
        jnp.dot(w1_ref[...], de, preferred_element_type=jnp.float32)
        + b1_ref[...], 0.0)
    di = jnp.maximum(
        jnp.dot(w2_ref[...], h, preferred_element_type=jnp.float32)
        + b2_ref[...], 0.0)
    tp = (jnp.dot(mi_ref[...], g_ref[...], preferred_element_type=jnp.float32)
          + jnp.dot(wtd_ref[...], di, preferred_element_type=jnp.float32)
          + bt1_ref[...])
    t = jnp.maximum(tp, 0.0)
    out_ref[...] = (jnp.dot(wt2_ref[...], t, preferred_element_type=jnp.float32)
                    + bt2_ref[...])


def _tc_dense(emb, price2, wp, bp2, w1, b12, w2, b22, m_int, wt1d, bt12,
              wt2, bt22):
    nblk = _B // _BBLK
    full = lambda a: pl.BlockSpec(a.shape, lambda i: tuple(0 for _ in a.shape))
    out2 = pl.pallas_call(
        _dense_body,
        grid=(nblk,),
        in_specs=[
            pl.BlockSpec((_F, _BBLK, _D), lambda i: (0, i, 0)),
            pl.BlockSpec((1, _BBLK), lambda i: (0, i)),
            full(wp), full(bp2), full(w1), full(b12),
            full(w2), full(b22), full(m_int),
            full(wt1d), full(bt12), full(wt2), full(bt22),
        ],
        out_specs=pl.BlockSpec((1, _BBLK), lambda i: (0, i)),
        out_shape=jax.ShapeDtypeStruct((1, _B), jnp.float32),
        scratch_shapes=[
            pltpu.VMEM((_NF, _D, _BBLK), jnp.float32),
            pltpu.VMEM((_NPAIR, _BBLK), jnp.float32),
        ],
    )(emb, price2, wp, bp2, w1, b12, w2, b22, m_int, wt1d, bt12, wt2, bt22)
    return out2


def kernel(x_cat, price, tables, W_price, b_price, W1, b1, W2, b2,
           Wt1, bt1, Wt2, bt2):
    offs = (jnp.arange(_F, dtype=jnp.int32) * _V)[:, None]
    idx3 = (x_cat.astype(jnp.int32) + offs).reshape(_NW, _NT, _TR)
    table_flat = tables.reshape(_F * _V, _D)
    emb_flat = _sc_gather(table_flat, idx3)
    emb = emb_flat.reshape(_F, _B, _D)
    out2 = _tc_dense(
        emb,
        price.reshape(1, _B),
        W_price,
        b_price.reshape(_D, 1),
        W1, b1.reshape(-1, 1),
        W2, b2.reshape(-1, 1),
        Wt1[:, :_NPAIR],
        Wt1[:, _NPAIR:],
        bt1.reshape(-1, 1),
        Wt2,
        bt2.reshape(1, 1),
    )
    return out2.reshape(_B)
